# Initial kernel scaffold; baseline (speedup 1.0000x reference)
import jax
import jax.numpy as jnp
from jax.experimental import pallas as pl


def kernel(attention_weights, mask):
    raise NotImplementedError("write your pallas kernel here")


# speedup vs baseline: 4.5532x; 4.5532x over previous
import functools

import jax
import jax.numpy as jnp
from jax import lax
from jax.experimental import pallas as pl
from jax.experimental.pallas import tpu as pltpu
from jax.experimental.pallas import tpu_sc as plsc

B = 128
L = 32768
TOPK_PCT = 0.2
TARGET = 0.95

NW = 32
ROWS_PER_W = B // NW
NBINS = 4096


def _stage1_body(aw_ref, mask_ref, valid_ref, k_ref, acc_ref):
    i = pl.program_id(0)
    m = mask_ref[...]
    a = aw_ref[...]
    valid_ref[...] = jnp.where(m, jnp.float32(0.0), a)
    part = jnp.sum(jnp.logical_not(m).astype(jnp.int32), axis=1, keepdims=True)

    @pl.when(i == 0)
    def _():
        acc_ref[...] = part

    @pl.when(i > 0)
    def _():
        acc_ref[...] = acc_ref[...] + part

    @pl.when(i == pl.num_programs(0) - 1)
    def _():
        n = acc_ref[...].astype(jnp.float32)
        k_ref[...] = jnp.maximum(
            jnp.int32(1), (n * jnp.float32(TOPK_PCT)).astype(jnp.int32))


def _stage1(aw, mask):
    blk = 2048
    grid = (L // blk,)
    return pl.pallas_call(
        _stage1_body,
        grid=grid,
        in_specs=[
            pl.BlockSpec((B, blk), lambda i: (0, i)),
            pl.BlockSpec((B, blk), lambda i: (0, i)),
        ],
        out_specs=[
            pl.BlockSpec((B, blk), lambda i: (0, i)),
            pl.BlockSpec((B, 1), lambda i: (0, 0)),
        ],
        out_shape=[
            jax.ShapeDtypeStruct((B, L), jnp.float32),
            jax.ShapeDtypeStruct((B, 1), jnp.int32),
        ],
        scratch_shapes=[pltpu.VMEM((B, 1), jnp.int32)],
    )(aw, mask)


def _scan_hist(cnt_ref, sum_ref, nbins, need):
    nblk = nbins // 16
    lane = lax.iota(jnp.int32, 16)

    def body(i, carry):
        c_acc, s_acc, b_sel, c_ab, s_ab, found = carry
        blk = nblk - 1 - i
        cv = cnt_ref[pl.ds(blk * 16, 16)]
        sv = sum_ref[pl.ds(blk * 16, 16)]
        rc = lax.rev(cv, (0,))
        rs = lax.rev(sv, (0,))
        ccum = jnp.cumsum(rc)
        scum = jnp.cumsum(rs)
        cross = (c_acc + ccum) >= need
        hit = jnp.any(cross)
        jv = plsc.all_reduce_ffs(cross)
        j_s = jv if jv.ndim == 0 else jnp.max(jv)
        sel = lane == j_s
        ccum_j = jnp.sum(jnp.where(sel, ccum, 0))
        rc_j = jnp.sum(jnp.where(sel, rc, 0))
        scum_j = jnp.sum(jnp.where(sel, scum, jnp.float32(0.0)))
        rs_j = jnp.sum(jnp.where(sel, rs, jnp.float32(0.0)))
        tot_c = jnp.sum(cv)
        tot_s = jnp.sum(sv)
        use = jnp.logical_and(jnp.logical_not(found), hit)
        nf = jnp.logical_or(found, hit)
        return (
            jnp.where(nf, c_acc, c_acc + tot_c),
            jnp.where(nf, s_acc, s_acc + tot_s),
            jnp.where(use, blk * 16 + 15 - j_s, b_sel),
            jnp.where(use, c_acc + ccum_j - rc_j, c_ab),
            jnp.where(use, s_acc + scum_j - rs_j, s_ab),
            nf,
        )

    init = (jnp.int32(0), jnp.float32(0.0), jnp.int32(0), jnp.int32(0),
            jnp.float32(0.0), jnp.bool_(False))
    _, _, b_sel, c_ab, s_ab, _ = lax.fori_loop(0, nblk, body, init)
    return b_sel, c_ab, s_ab


def _sc_body(valid_hbm, k_hbm, out_hbm, vals, cnt_h, sum_h, cnt3, sum3, kv,
             outv):
    wid = lax.axis_index("s") * 2 + lax.axis_index("c")
    lane = lax.iota(jnp.int32, 16)
    ones_i = jnp.full((16,), 1, jnp.int32)
    zeros_i = jnp.full((16,), 0, jnp.int32)
    zeros_f = jnp.full((16,), 0.0, jnp.float32)

    pltpu.sync_copy(k_hbm, kv)
    outv[...] = zeros_f

    kbase = (wid // 4) * 16
    kvec = kv[pl.ds(kbase, 16)]

    for j in range(ROWS_PER_W):
        row = wid * ROWS_PER_W + j
        pltpu.sync_copy(valid_hbm.at[row], vals)

        klane = (wid % 4) * 4 + j
        k = jnp.sum(jnp.where(lane == klane, kvec, 0))

        def clr(t, _):
            base = t * 64
            for u in range(4):
                cnt_h[pl.ds(base + u * 16, 16)] = zeros_i
                sum_h[pl.ds(base + u * 16, 16)] = zeros_f
            return 0

        lax.fori_loop(0, NBINS // 64, clr, 0)

        def p1(t, _):
            base = t * 128
            for u in range(8):
                v = vals[pl.ds(base + u * 16, 16)]
                bits = plsc.bitcast(v, jnp.int32)
                b1 = lax.shift_right_logical(bits, 18)
                plsc.addupdate_scatter(cnt_h, [b1], ones_i)
                plsc.addupdate_scatter(sum_h, [b1], v)
            return 0

        lax.fori_loop(0, L // 128, p1, 0)
        b1, c1, s1 = _scan_hist(cnt_h, sum_h, NBINS, k)
        r = k - c1
        s_above = s1

        lax.fori_loop(0, NBINS // 64, clr, 0)

        def p2(t, _):
            base = t * 128
            for u in range(8):
                v = vals[pl.ds(base + u * 16, 16)]
                bits = plsc.bitcast(v, jnp.int32)
                m = lax.shift_right_logical(bits, 18) == b1
                b2 = jnp.bitwise_and(lax.shift_right_logical(bits, 6),
                                     jnp.int32(0xFFF))
                plsc.addupdate_scatter(cnt_h, [b2], ones_i, mask=m)
                plsc.addupdate_scatter(sum_h, [b2], v, mask=m)
            return 0

        lax.fori_loop(0, L // 128, p2, 0)
        b2, c2, s2 = _scan_hist(cnt_h, sum_h, NBINS, r)
        r = r - c2
        s_above = s_above + s2

        for u in range(4):
            cnt3[pl.ds(u * 16, 16)] = zeros_i
            sum3[pl.ds(u * 16, 16)] = zeros_f
        pref = jnp.bitwise_or(lax.shift_left(b1, 12), b2)

        def p3(t, _):
            base = t * 128
            for u in range(8):
                v = vals[pl.ds(base + u * 16, 16)]
                bits = plsc.bitcast(v, jnp.int32)
                m = lax.shift_right_logical(bits, 6) == pref
                b3 = jnp.bitwise_and(bits, jnp.int32(0x3F))
                plsc.addupdate_scatter(cnt3, [b3], ones_i, mask=m)
                plsc.addupdate_scatter(sum3, [b3], v, mask=m)
            return 0

        lax.fori_loop(0, L // 128, p3, 0)
        b3, c3, s3 = _scan_hist(cnt3, sum3, 64, r)
        r = r - c3
        s_above = s_above + s3

        vbits = jnp.bitwise_or(
            jnp.bitwise_or(lax.shift_left(b1, 18), lax.shift_left(b2, 6)), b3)
        vf = plsc.bitcast(jnp.full((16,), 1, jnp.int32) * vbits, jnp.float32)
        val = jnp.max(vf)
        conc = s_above + r.astype(jnp.float32) * val
        outv[...] = jnp.where(lane == j, conc, outv[...])

    pltpu.sync_copy(outv, out_hbm.at[wid])


def _stage2(valid, karr):
    mesh = plsc.VectorSubcoreMesh(core_axis_name="c", subcore_axis_name="s")
    f = functools.partial(
        pl.kernel,
        out_type=jax.ShapeDtypeStruct((NW, 16), jnp.float32),
        mesh=mesh,
        scratch_types=[
            pltpu.VMEM((L,), jnp.float32),
            pltpu.VMEM((NBINS,), jnp.int32),
            pltpu.VMEM((NBINS,), jnp.float32),
            pltpu.VMEM((64,), jnp.int32),
            pltpu.VMEM((64,), jnp.float32),
            pltpu.VMEM((B,), jnp.int32),
            pltpu.VMEM((16,), jnp.float32),
        ],
        compiler_params=pltpu.CompilerParams(needs_layout_passes=False),
    )(_sc_body)
    return f(valid, karr)


def _stage3_body(conc_ref, loss_ref, cmean_ref):
    c = conc_ref[...]
    colmask = lax.broadcasted_iota(jnp.int32, (NW, 16), 1) < ROWS_PER_W
    cm = jnp.where(colmask, c, jnp.float32(0.0))
    cmean_ref[...] = jnp.sum(cm, keepdims=True) / jnp.float32(B)
    loss = jnp.maximum(jnp.float32(TARGET) - c, jnp.float32(0.0))
    loss_ref[...] = jnp.sum(jnp.where(colmask, loss, jnp.float32(0.0)),
                            keepdims=True) / jnp.float32(B)


def _stage3(conc2d):
    return pl.pallas_call(
        _stage3_body,
        out_shape=[
            jax.ShapeDtypeStruct((1, 1), jnp.float32),
            jax.ShapeDtypeStruct((1, 1), jnp.float32),
        ],
    )(conc2d)


def kernel(attention_weights, mask):
    valid, k2d = _stage1(attention_weights, mask)
    conc2d = _stage2(valid, k2d.reshape(B))
    loss2d, cmean2d = _stage3(conc2d)
    return loss2d[0, 0], cmean2d[0, 0]


# speedup vs baseline: 4.9154x; 1.0795x over previous
import functools

import jax
import jax.numpy as jnp
from jax import lax
from jax.experimental import pallas as pl
from jax.experimental.pallas import tpu as pltpu
from jax.experimental.pallas import tpu_sc as plsc

B = 128
L = 32768
TOPK_PCT = 0.2
TARGET = 0.95

NW = 32
ROWS_PER_W = B // NW
NBINS = 4096


def _stage1_body(aw_ref, mask_ref, valid_ref, k_ref, acc_ref):
    i = pl.program_id(0)
    m = mask_ref[...]
    a = aw_ref[...]
    valid_ref[...] = jnp.where(m, jnp.float32(0.0), a)
    part = jnp.sum(jnp.logical_not(m).astype(jnp.int32), axis=1, keepdims=True)

    @pl.when(i == 0)
    def _():
        acc_ref[...] = part

    @pl.when(i > 0)
    def _():
        acc_ref[...] = acc_ref[...] + part

    @pl.when(i == pl.num_programs(0) - 1)
    def _():
        n = acc_ref[...].astype(jnp.float32)
        k_ref[...] = jnp.maximum(
            jnp.int32(1), (n * jnp.float32(TOPK_PCT)).astype(jnp.int32))


def _stage1(aw, mask):
    blk = 2048
    grid = (L // blk,)
    return pl.pallas_call(
        _stage1_body,
        grid=grid,
        in_specs=[
            pl.BlockSpec((B, blk), lambda i: (0, i)),
            pl.BlockSpec((B, blk), lambda i: (0, i)),
        ],
        out_specs=[
            pl.BlockSpec((B, blk), lambda i: (0, i)),
            pl.BlockSpec((B, 1), lambda i: (0, 0)),
        ],
        out_shape=[
            jax.ShapeDtypeStruct((B, L), jnp.float32),
            jax.ShapeDtypeStruct((B, 1), jnp.int32),
        ],
        scratch_shapes=[pltpu.VMEM((B, 1), jnp.int32)],
    )(aw, mask)


def _resolve16(vc, vs, need):
    lane = lax.iota(jnp.int32, 16)
    rc = lax.rev(vc, (0,))
    rs = lax.rev(vs, (0,))
    ccum = jnp.cumsum(rc)
    scum = jnp.cumsum(rs)
    cross = ccum >= need
    jv = plsc.all_reduce_ffs(cross)
    j_s = jv if jv.ndim == 0 else jnp.max(jv)
    sel = lane == j_s
    c_ab = jnp.sum(jnp.where(sel, ccum - rc, 0))
    s_ab = jnp.sum(jnp.where(sel, scum - rs, jnp.float32(0.0)))
    return 15 - j_s, c_ab, s_ab


def _scan_hist4096(cnt_ref, sum_ref, mid_c, mid_s, need):
    lane = lax.iota(jnp.int32, 16)
    lane16 = lane * 16

    def build(j, _):
        base = j * 256
        vc = jnp.full((16,), 0, jnp.int32)
        vs = jnp.full((16,), 0.0, jnp.float32)
        for w in range(16):
            idx = lane16 + (base + w)
            vc = vc + plsc.load_gather(cnt_ref, [idx])
            vs = vs + plsc.load_gather(sum_ref, [idx])
        mid_c[pl.ds(j * 16, 16)] = vc
        mid_s[pl.ds(j * 16, 16)] = vs
        return 0

    lax.fori_loop(0, 16, build, 0)

    tc_v = jnp.full((16,), 0, jnp.int32)
    ts_v = jnp.full((16,), 0.0, jnp.float32)
    for w in range(16):
        idx = lane16 + w
        tc_v = tc_v + plsc.load_gather(mid_c, [idx])
        ts_v = ts_v + plsc.load_gather(mid_s, [idx])

    g, c1, s1 = _resolve16(tc_v, ts_v, need)
    r1 = need - c1
    mv_c = mid_c[pl.ds(g * 16, 16)]
    mv_s = mid_s[pl.ds(g * 16, 16)]
    m, c2, s2 = _resolve16(mv_c, mv_s, r1)
    r2 = r1 - c2
    fblk = g * 16 + m
    fv_c = cnt_ref[pl.ds(fblk * 16, 16)]
    fv_s = sum_ref[pl.ds(fblk * 16, 16)]
    p, c3, s3 = _resolve16(fv_c, fv_s, r2)
    return fblk * 16 + p, c1 + c2 + c3, s1 + s2 + s3


def _scan_hist64(cnt_ref, sum_ref, need):
    lane = lax.iota(jnp.int32, 16)
    tc_v = jnp.full((16,), 0, jnp.int32)
    ts_v = jnp.full((16,), 0.0, jnp.float32)
    for i in range(4):
        ci = jnp.sum(cnt_ref[pl.ds(i * 16, 16)])
        si = jnp.sum(sum_ref[pl.ds(i * 16, 16)])
        tc_v = jnp.where(lane == i, ci, tc_v)
        ts_v = jnp.where(lane == i, si, ts_v)
    g, c1, s1 = _resolve16(tc_v, ts_v, need)
    r1 = need - c1
    p, c2, s2 = _resolve16(cnt_ref[pl.ds(g * 16, 16)],
                           sum_ref[pl.ds(g * 16, 16)], r1)
    return g * 16 + p, c1 + c2, s1 + s2


def _sc_body(valid_hbm, k_hbm, out_hbm, vals, cnt_h, sum_h, cnt3, sum3, kv,
             outv, mid_c, mid_s):
    wid = lax.axis_index("s") * 2 + lax.axis_index("c")
    lane = lax.iota(jnp.int32, 16)
    ones_i = jnp.full((16,), 1, jnp.int32)
    zeros_i = jnp.full((16,), 0, jnp.int32)
    zeros_f = jnp.full((16,), 0.0, jnp.float32)

    pltpu.sync_copy(k_hbm, kv)
    outv[...] = zeros_f

    kbase = (wid // 4) * 16
    kvec = kv[pl.ds(kbase, 16)]

    for j in range(ROWS_PER_W):
        row = wid * ROWS_PER_W + j
        pltpu.sync_copy(valid_hbm.at[row], vals)

        klane = (wid % 4) * 4 + j
        k = jnp.sum(jnp.where(lane == klane, kvec, 0))

        def clr(t, _):
            base = t * 64
            for u in range(4):
                cnt_h[pl.ds(base + u * 16, 16)] = zeros_i
                sum_h[pl.ds(base + u * 16, 16)] = zeros_f
            return 0

        lax.fori_loop(0, NBINS // 64, clr, 0)

        def p1(t, _):
            base = t * 128
            for u in range(8):
                v = vals[pl.ds(base + u * 16, 16)]
                bits = plsc.bitcast(v, jnp.int32)
                b1 = lax.shift_right_logical(bits, 18)
                plsc.addupdate_scatter(cnt_h, [b1], ones_i)
                plsc.addupdate_scatter(sum_h, [b1], v)
            return 0

        lax.fori_loop(0, L // 128, p1, 0)
        b1, c1, s1 = _scan_hist4096(cnt_h, sum_h, mid_c, mid_s, k)
        r = k - c1
        s_above = s1

        lax.fori_loop(0, NBINS // 64, clr, 0)

        def p2(t, _):
            base = t * 128
            for u in range(8):
                v = vals[pl.ds(base + u * 16, 16)]
                bits = plsc.bitcast(v, jnp.int32)
                m = lax.shift_right_logical(bits, 18) == b1
                b2 = jnp.bitwise_and(lax.shift_right_logical(bits, 6),
                                     jnp.int32(0xFFF))
                plsc.addupdate_scatter(cnt_h, [b2], ones_i, mask=m)
                plsc.addupdate_scatter(sum_h, [b2], v, mask=m)
            return 0

        lax.fori_loop(0, L // 128, p2, 0)
        b2, c2, s2 = _scan_hist4096(cnt_h, sum_h, mid_c, mid_s, r)
        r = r - c2
        s_above = s_above + s2

        for u in range(4):
            cnt3[pl.ds(u * 16, 16)] = zeros_i
            sum3[pl.ds(u * 16, 16)] = zeros_f
        pref = jnp.bitwise_or(lax.shift_left(b1, 12), b2)

        def p3(t, _):
            base = t * 128
            for u in range(8):
                v = vals[pl.ds(base + u * 16, 16)]
                bits = plsc.bitcast(v, jnp.int32)
                m = lax.shift_right_logical(bits, 6) == pref
                b3 = jnp.bitwise_and(bits, jnp.int32(0x3F))
                plsc.addupdate_scatter(cnt3, [b3], ones_i, mask=m)
                plsc.addupdate_scatter(sum3, [b3], v, mask=m)
            return 0

        lax.fori_loop(0, L // 128, p3, 0)
        b3, c3, s3 = _scan_hist64(cnt3, sum3, r)
        r = r - c3
        s_above = s_above + s3

        vbits = jnp.bitwise_or(
            jnp.bitwise_or(lax.shift_left(b1, 18), lax.shift_left(b2, 6)), b3)
        vf = plsc.bitcast(jnp.full((16,), 1, jnp.int32) * vbits, jnp.float32)
        val = jnp.max(vf)
        conc = s_above + r.astype(jnp.float32) * val
        outv[...] = jnp.where(lane == j, conc, outv[...])

    pltpu.sync_copy(outv, out_hbm.at[wid])


def _stage2(valid, karr):
    mesh = plsc.VectorSubcoreMesh(core_axis_name="c", subcore_axis_name="s")
    f = functools.partial(
        pl.kernel,
        out_type=jax.ShapeDtypeStruct((NW, 16), jnp.float32),
        mesh=mesh,
        scratch_types=[
            pltpu.VMEM((L,), jnp.float32),
            pltpu.VMEM((NBINS,), jnp.int32),
            pltpu.VMEM((NBINS,), jnp.float32),
            pltpu.VMEM((64,), jnp.int32),
            pltpu.VMEM((64,), jnp.float32),
            pltpu.VMEM((B,), jnp.int32),
            pltpu.VMEM((16,), jnp.float32),
            pltpu.VMEM((256,), jnp.int32),
            pltpu.VMEM((256,), jnp.float32),
        ],
        compiler_params=pltpu.CompilerParams(needs_layout_passes=False),
    )(_sc_body)
    return f(valid, karr)


def _stage3_body(conc_ref, loss_ref, cmean_ref):
    c = conc_ref[...]
    colmask = lax.broadcasted_iota(jnp.int32, (NW, 16), 1) < ROWS_PER_W
    cm = jnp.where(colmask, c, jnp.float32(0.0))
    cmean_ref[...] = jnp.sum(cm, keepdims=True) / jnp.float32(B)
    loss = jnp.maximum(jnp.float32(TARGET) - c, jnp.float32(0.0))
    loss_ref[...] = jnp.sum(jnp.where(colmask, loss, jnp.float32(0.0)),
                            keepdims=True) / jnp.float32(B)


def _stage3(conc2d):
    return pl.pallas_call(
        _stage3_body,
        out_shape=[
            jax.ShapeDtypeStruct((1, 1), jnp.float32),
            jax.ShapeDtypeStruct((1, 1), jnp.float32),
        ],
    )(conc2d)


def kernel(attention_weights, mask):
    valid, k2d = _stage1(attention_weights, mask)
    conc2d = _stage2(valid, k2d.reshape(B))
    loss2d, cmean2d = _stage3(conc2d)
    return loss2d[0, 0], cmean2d[0, 0]


# speedup vs baseline: 7.9660x; 1.6206x over previous
import functools

import jax
import jax.numpy as jnp
from jax import lax
from jax.experimental import pallas as pl
from jax.experimental.pallas import tpu as pltpu
from jax.experimental.pallas import tpu_sc as plsc

B = 128
L = 32768
TOPK_PCT = 0.2
TARGET = 0.95

NW = 32
ROWS_PER_W = B // NW
NBINS = 4096


def _stage1_body(aw_ref, mask_ref, valid_ref, k_ref, acc_ref):
    i = pl.program_id(0)
    m = mask_ref[...]
    a = aw_ref[...]
    valid_ref[...] = jnp.where(m, jnp.float32(0.0), a)
    part = jnp.sum(jnp.logical_not(m).astype(jnp.int32), axis=1, keepdims=True)

    @pl.when(i == 0)
    def _():
        acc_ref[...] = part

    @pl.when(i > 0)
    def _():
        acc_ref[...] = acc_ref[...] + part

    @pl.when(i == pl.num_programs(0) - 1)
    def _():
        n = acc_ref[...].astype(jnp.float32)
        k_ref[...] = jnp.maximum(
            jnp.int32(1), (n * jnp.float32(TOPK_PCT)).astype(jnp.int32))


def _stage1(aw, mask):
    blk = 2048
    grid = (L // blk,)
    return pl.pallas_call(
        _stage1_body,
        grid=grid,
        in_specs=[
            pl.BlockSpec((B, blk), lambda i: (0, i)),
            pl.BlockSpec((B, blk), lambda i: (0, i)),
        ],
        out_specs=[
            pl.BlockSpec((B, blk), lambda i: (0, i)),
            pl.BlockSpec((B, 1), lambda i: (0, 0)),
        ],
        out_shape=[
            jax.ShapeDtypeStruct((B, L), jnp.float32),
            jax.ShapeDtypeStruct((B, 1), jnp.int32),
        ],
        scratch_shapes=[pltpu.VMEM((B, 1), jnp.int32)],
    )(aw, mask)


def _resolve16(vc, vs, need):
    lane = lax.iota(jnp.int32, 16)
    rc = lax.rev(vc, (0,))
    rs = lax.rev(vs, (0,))
    ccum = jnp.cumsum(rc)
    scum = jnp.cumsum(rs)
    cross = ccum >= need
    jv = plsc.all_reduce_ffs(cross)
    j_s = jv if jv.ndim == 0 else jnp.max(jv)
    sel = lane == j_s
    c_ab = jnp.sum(jnp.where(sel, ccum - rc, 0))
    s_ab = jnp.sum(jnp.where(sel, scum - rs, jnp.float32(0.0)))
    return 15 - j_s, c_ab, s_ab


def _scan_hist4096(cnt_ref, sum_ref, mid_c, mid_s, need):
    lane = lax.iota(jnp.int32, 16)
    lane16 = lane * 16

    @plsc.parallel_loop(0, 16, step=1)
    def _build(j):
        base = j * 256
        vc = jnp.full((16,), 0, jnp.int32)
        vs = jnp.full((16,), 0.0, jnp.float32)
        for w in range(16):
            idx = lane16 + (base + w)
            vc = vc + plsc.load_gather(cnt_ref, [idx])
            vs = vs + plsc.load_gather(sum_ref, [idx])
        mid_c[pl.ds(j * 16, 16)] = vc
        mid_s[pl.ds(j * 16, 16)] = vs

    tc_v = jnp.full((16,), 0, jnp.int32)
    ts_v = jnp.full((16,), 0.0, jnp.float32)
    for w in range(16):
        idx = lane16 + w
        tc_v = tc_v + plsc.load_gather(mid_c, [idx])
        ts_v = ts_v + plsc.load_gather(mid_s, [idx])

    g, c1, s1 = _resolve16(tc_v, ts_v, need)
    r1 = need - c1
    mv_c = mid_c[pl.ds(g * 16, 16)]
    mv_s = mid_s[pl.ds(g * 16, 16)]
    m, c2, s2 = _resolve16(mv_c, mv_s, r1)
    r2 = r1 - c2
    fblk = g * 16 + m
    fv_c = cnt_ref[pl.ds(fblk * 16, 16)]
    fv_s = sum_ref[pl.ds(fblk * 16, 16)]
    p, c3, s3 = _resolve16(fv_c, fv_s, r2)
    return fblk * 16 + p, c1 + c2 + c3, s1 + s2 + s3


def _scan_hist64(cnt_ref, sum_ref, need):
    lane = lax.iota(jnp.int32, 16)
    tc_v = jnp.full((16,), 0, jnp.int32)
    ts_v = jnp.full((16,), 0.0, jnp.float32)
    for i in range(4):
        ci = jnp.sum(cnt_ref[pl.ds(i * 16, 16)])
        si = jnp.sum(sum_ref[pl.ds(i * 16, 16)])
        tc_v = jnp.where(lane == i, ci, tc_v)
        ts_v = jnp.where(lane == i, si, ts_v)
    g, c1, s1 = _resolve16(tc_v, ts_v, need)
    r1 = need - c1
    p, c2, s2 = _resolve16(cnt_ref[pl.ds(g * 16, 16)],
                           sum_ref[pl.ds(g * 16, 16)], r1)
    return g * 16 + p, c1 + c2, s1 + s2


def _sc_body(valid_hbm, k_hbm, out_hbm, vals, cnt_h, sum_h, cnt3, sum3, kv,
             outv, mid_c, mid_s):
    wid = lax.axis_index("s") * 2 + lax.axis_index("c")
    lane = lax.iota(jnp.int32, 16)
    ones_i = jnp.full((16,), 1, jnp.int32)
    zeros_i = jnp.full((16,), 0, jnp.int32)
    zeros_f = jnp.full((16,), 0.0, jnp.float32)

    pltpu.sync_copy(k_hbm, kv)
    outv[...] = zeros_f

    kbase = (wid // 4) * 16
    kvec = kv[pl.ds(kbase, 16)]

    for j in range(ROWS_PER_W):
        row = wid * ROWS_PER_W + j
        pltpu.sync_copy(valid_hbm.at[row], vals)

        klane = (wid % 4) * 4 + j
        k = jnp.sum(jnp.where(lane == klane, kvec, 0))

        @plsc.parallel_loop(0, NBINS // 64, step=1)
        def _clr1(t):
            base = t * 64
            for u in range(4):
                cnt_h[pl.ds(base + u * 16, 16)] = zeros_i
                sum_h[pl.ds(base + u * 16, 16)] = zeros_f

        @plsc.parallel_loop(0, L // 128, step=1)
        def _p1(t):
            base = t * 128
            for u in range(8):
                v = vals[pl.ds(base + u * 16, 16)]
                bits = plsc.bitcast(v, jnp.int32)
                b1v = lax.shift_right_logical(bits, 18)
                plsc.addupdate_scatter(cnt_h, [b1v], ones_i)
                plsc.addupdate_scatter(sum_h, [b1v], v)
        b1, c1, s1 = _scan_hist4096(cnt_h, sum_h, mid_c, mid_s, k)
        r = k - c1
        s_above = s1

        @plsc.parallel_loop(0, NBINS // 64, step=1)
        def _clr2(t):
            base = t * 64
            for u in range(4):
                cnt_h[pl.ds(base + u * 16, 16)] = zeros_i
                sum_h[pl.ds(base + u * 16, 16)] = zeros_f

        @plsc.parallel_loop(0, L // 128, step=1)
        def _p2(t):
            base = t * 128
            for u in range(8):
                v = vals[pl.ds(base + u * 16, 16)]
                bits = plsc.bitcast(v, jnp.int32)
                m = lax.shift_right_logical(bits, 18) == b1
                b2v = jnp.bitwise_and(lax.shift_right_logical(bits, 6),
                                      jnp.int32(0xFFF))
                plsc.addupdate_scatter(cnt_h, [b2v], ones_i, mask=m)
                plsc.addupdate_scatter(sum_h, [b2v], v, mask=m)
        b2, c2, s2 = _scan_hist4096(cnt_h, sum_h, mid_c, mid_s, r)
        r = r - c2
        s_above = s_above + s2

        for u in range(4):
            cnt3[pl.ds(u * 16, 16)] = zeros_i
            sum3[pl.ds(u * 16, 16)] = zeros_f
        pref = jnp.bitwise_or(lax.shift_left(b1, 12), b2)

        @plsc.parallel_loop(0, L // 128, step=1)
        def _p3(t):
            base = t * 128
            for u in range(8):
                v = vals[pl.ds(base + u * 16, 16)]
                bits = plsc.bitcast(v, jnp.int32)
                m = lax.shift_right_logical(bits, 6) == pref
                b3v = jnp.bitwise_and(bits, jnp.int32(0x3F))
                plsc.addupdate_scatter(cnt3, [b3v], ones_i, mask=m)
                plsc.addupdate_scatter(sum3, [b3v], v, mask=m)
        b3, c3, s3 = _scan_hist64(cnt3, sum3, r)
        r = r - c3
        s_above = s_above + s3

        vbits = jnp.bitwise_or(
            jnp.bitwise_or(lax.shift_left(b1, 18), lax.shift_left(b2, 6)), b3)
        vf = plsc.bitcast(jnp.full((16,), 1, jnp.int32) * vbits, jnp.float32)
        val = jnp.max(vf)
        conc = s_above + r.astype(jnp.float32) * val
        outv[...] = jnp.where(lane == j, conc, outv[...])

    pltpu.sync_copy(outv, out_hbm.at[wid])


def _stage2(valid, karr):
    mesh = plsc.VectorSubcoreMesh(core_axis_name="c", subcore_axis_name="s")
    f = functools.partial(
        pl.kernel,
        out_type=jax.ShapeDtypeStruct((NW, 16), jnp.float32),
        mesh=mesh,
        scratch_types=[
            pltpu.VMEM((L,), jnp.float32),
            pltpu.VMEM((NBINS,), jnp.int32),
            pltpu.VMEM((NBINS,), jnp.float32),
            pltpu.VMEM((64,), jnp.int32),
            pltpu.VMEM((64,), jnp.float32),
            pltpu.VMEM((B,), jnp.int32),
            pltpu.VMEM((16,), jnp.float32),
            pltpu.VMEM((256,), jnp.int32),
            pltpu.VMEM((256,), jnp.float32),
        ],
        compiler_params=pltpu.CompilerParams(needs_layout_passes=False),
    )(_sc_body)
    return f(valid, karr)


def _stage3_body(conc_ref, loss_ref, cmean_ref):
    c = conc_ref[...]
    colmask = lax.broadcasted_iota(jnp.int32, (NW, 16), 1) < ROWS_PER_W
    cm = jnp.where(colmask, c, jnp.float32(0.0))
    cmean_ref[...] = jnp.sum(cm, keepdims=True) / jnp.float32(B)
    loss = jnp.maximum(jnp.float32(TARGET) - c, jnp.float32(0.0))
    loss_ref[...] = jnp.sum(jnp.where(colmask, loss, jnp.float32(0.0)),
                            keepdims=True) / jnp.float32(B)


def _stage3(conc2d):
    return pl.pallas_call(
        _stage3_body,
        out_shape=[
            jax.ShapeDtypeStruct((1, 1), jnp.float32),
            jax.ShapeDtypeStruct((1, 1), jnp.float32),
        ],
    )(conc2d)


def kernel(attention_weights, mask):
    valid, k2d = _stage1(attention_weights, mask)
    conc2d = _stage2(valid, k2d.reshape(B))
    loss2d, cmean2d = _stage3(conc2d)
    return loss2d[0, 0], cmean2d[0, 0]


# speedup vs baseline: 7.9722x; 1.0008x over previous
import functools

import jax
import jax.numpy as jnp
from jax import lax
from jax.experimental import pallas as pl
from jax.experimental.pallas import tpu as pltpu
from jax.experimental.pallas import tpu_sc as plsc

B = 128
L = 32768
TOPK_PCT = 0.2
TARGET = 0.95

NW = 32
ROWS_PER_W = B // NW
NBINS = 4096


def _stage1_body(aw_ref, mask_ref, valid_ref, k_ref, acc_ref):
    i = pl.program_id(0)
    m = mask_ref[...]
    a = aw_ref[...]
    valid_ref[...] = jnp.where(m, jnp.float32(0.0), a)
    part = jnp.sum(jnp.logical_not(m).astype(jnp.int32), axis=1, keepdims=True)

    @pl.when(i == 0)
    def _():
        acc_ref[...] = part

    @pl.when(i > 0)
    def _():
        acc_ref[...] = acc_ref[...] + part

    @pl.when(i == pl.num_programs(0) - 1)
    def _():
        n = acc_ref[...].astype(jnp.float32)
        k_ref[...] = jnp.maximum(
            jnp.int32(1), (n * jnp.float32(TOPK_PCT)).astype(jnp.int32))


def _stage1(aw, mask):
    blk = 2048
    grid = (L // blk,)
    return pl.pallas_call(
        _stage1_body,
        grid=grid,
        in_specs=[
            pl.BlockSpec((B, blk), lambda i: (0, i)),
            pl.BlockSpec((B, blk), lambda i: (0, i)),
        ],
        out_specs=[
            pl.BlockSpec((B, blk), lambda i: (0, i)),
            pl.BlockSpec((B, 1), lambda i: (0, 0)),
        ],
        out_shape=[
            jax.ShapeDtypeStruct((B, L), jnp.float32),
            jax.ShapeDtypeStruct((B, 1), jnp.int32),
        ],
        scratch_shapes=[pltpu.VMEM((B, 1), jnp.int32)],
    )(aw, mask)


def _resolve16(vc, vs, need):
    lane = lax.iota(jnp.int32, 16)
    rc = lax.rev(vc, (0,))
    rs = lax.rev(vs, (0,))
    ccum = jnp.cumsum(rc)
    scum = jnp.cumsum(rs)
    cross = ccum >= need
    jv = plsc.all_reduce_ffs(cross)
    j_s = jv if jv.ndim == 0 else jnp.max(jv)
    sel = lane == j_s
    c_ab = jnp.sum(jnp.where(sel, ccum - rc, 0))
    s_ab = jnp.sum(jnp.where(sel, scum - rs, jnp.float32(0.0)))
    return 15 - j_s, c_ab, s_ab


def _scan_hist4096(cnt_ref, sum_ref, mid_c, mid_s, need):
    lane = lax.iota(jnp.int32, 16)
    lane16 = lane * 16

    @plsc.parallel_loop(0, 16, step=1)
    def _build(j):
        base = j * 256
        vc = jnp.full((16,), 0, jnp.int32)
        vs = jnp.full((16,), 0.0, jnp.float32)
        for w in range(16):
            idx = lane16 + (base + w)
            vc = vc + plsc.load_gather(cnt_ref, [idx])
            vs = vs + plsc.load_gather(sum_ref, [idx])
        mid_c[pl.ds(j * 16, 16)] = vc
        mid_s[pl.ds(j * 16, 16)] = vs

    tc_v = jnp.full((16,), 0, jnp.int32)
    ts_v = jnp.full((16,), 0.0, jnp.float32)
    for w in range(16):
        idx = lane16 + w
        tc_v = tc_v + plsc.load_gather(mid_c, [idx])
        ts_v = ts_v + plsc.load_gather(mid_s, [idx])

    g, c1, s1 = _resolve16(tc_v, ts_v, need)
    r1 = need - c1
    mv_c = mid_c[pl.ds(g * 16, 16)]
    mv_s = mid_s[pl.ds(g * 16, 16)]
    m, c2, s2 = _resolve16(mv_c, mv_s, r1)
    r2 = r1 - c2
    fblk = g * 16 + m
    fv_c = cnt_ref[pl.ds(fblk * 16, 16)]
    fv_s = sum_ref[pl.ds(fblk * 16, 16)]
    p, c3, s3 = _resolve16(fv_c, fv_s, r2)
    return fblk * 16 + p, c1 + c2 + c3, s1 + s2 + s3


def _scan_hist64(cnt_ref, sum_ref, need):
    lane = lax.iota(jnp.int32, 16)
    tc_v = jnp.full((16,), 0, jnp.int32)
    ts_v = jnp.full((16,), 0.0, jnp.float32)
    for i in range(4):
        ci = jnp.sum(cnt_ref[pl.ds(i * 16, 16)])
        si = jnp.sum(sum_ref[pl.ds(i * 16, 16)])
        tc_v = jnp.where(lane == i, ci, tc_v)
        ts_v = jnp.where(lane == i, si, ts_v)
    g, c1, s1 = _resolve16(tc_v, ts_v, need)
    r1 = need - c1
    p, c2, s2 = _resolve16(cnt_ref[pl.ds(g * 16, 16)],
                           sum_ref[pl.ds(g * 16, 16)], r1)
    return g * 16 + p, c1 + c2, s1 + s2


def _sc_body(valid_hbm, k_hbm, out_hbm, vals, cnt_h, sum_h, cnt3, sum3, kv,
             outv, mid_c, mid_s):
    wid = lax.axis_index("s") * 2 + lax.axis_index("c")
    lane = lax.iota(jnp.int32, 16)
    ones_i = jnp.full((16,), 1, jnp.int32)
    zeros_i = jnp.full((16,), 0, jnp.int32)
    zeros_f = jnp.full((16,), 0.0, jnp.float32)

    pltpu.sync_copy(k_hbm, kv)
    outv[...] = zeros_f

    kbase = (wid // 4) * 16
    kvec = kv[pl.ds(kbase, 16)]

    for j in range(ROWS_PER_W):
        row = wid * ROWS_PER_W + j
        pltpu.sync_copy(valid_hbm.at[row], vals)

        klane = (wid % 4) * 4 + j
        k = jnp.sum(jnp.where(lane == klane, kvec, 0))

        @plsc.parallel_loop(0, NBINS // 64, step=1)
        def _clr1(t):
            base = t * 64
            for u in range(4):
                cnt_h[pl.ds(base + u * 16, 16)] = zeros_i
                sum_h[pl.ds(base + u * 16, 16)] = zeros_f

        @plsc.parallel_loop(0, L // 16, step=1, unroll=8)
        def _p1(t):
            v = vals[pl.ds(t * 16, 16)]
            bits = plsc.bitcast(v, jnp.int32)
            b1v = lax.shift_right_logical(bits, 18)
            plsc.addupdate_scatter(cnt_h, [b1v], ones_i)
            plsc.addupdate_scatter(sum_h, [b1v], v)
        b1, c1, s1 = _scan_hist4096(cnt_h, sum_h, mid_c, mid_s, k)
        r = k - c1
        s_above = s1

        @plsc.parallel_loop(0, NBINS // 64, step=1)
        def _clr2(t):
            base = t * 64
            for u in range(4):
                cnt_h[pl.ds(base + u * 16, 16)] = zeros_i
                sum_h[pl.ds(base + u * 16, 16)] = zeros_f

        @plsc.parallel_loop(0, L // 16, step=1, unroll=8)
        def _p2(t):
            v = vals[pl.ds(t * 16, 16)]
            bits = plsc.bitcast(v, jnp.int32)
            m = lax.shift_right_logical(bits, 18) == b1
            b2v = jnp.bitwise_and(lax.shift_right_logical(bits, 6),
                                  jnp.int32(0xFFF))
            plsc.addupdate_scatter(cnt_h, [b2v], ones_i, mask=m)
            plsc.addupdate_scatter(sum_h, [b2v], v, mask=m)
        b2, c2, s2 = _scan_hist4096(cnt_h, sum_h, mid_c, mid_s, r)
        r = r - c2
        s_above = s_above + s2

        for u in range(4):
            cnt3[pl.ds(u * 16, 16)] = zeros_i
            sum3[pl.ds(u * 16, 16)] = zeros_f
        pref = jnp.bitwise_or(lax.shift_left(b1, 12), b2)

        @plsc.parallel_loop(0, L // 16, step=1, unroll=8)
        def _p3(t):
            v = vals[pl.ds(t * 16, 16)]
            bits = plsc.bitcast(v, jnp.int32)
            m = lax.shift_right_logical(bits, 6) == pref
            b3v = jnp.bitwise_and(bits, jnp.int32(0x3F))
            plsc.addupdate_scatter(cnt3, [b3v], ones_i, mask=m)
            plsc.addupdate_scatter(sum3, [b3v], v, mask=m)
        b3, c3, s3 = _scan_hist64(cnt3, sum3, r)
        r = r - c3
        s_above = s_above + s3

        vbits = jnp.bitwise_or(
            jnp.bitwise_or(lax.shift_left(b1, 18), lax.shift_left(b2, 6)), b3)
        vf = plsc.bitcast(jnp.full((16,), 1, jnp.int32) * vbits, jnp.float32)
        val = jnp.max(vf)
        conc = s_above + r.astype(jnp.float32) * val
        outv[...] = jnp.where(lane == j, conc, outv[...])

    pltpu.sync_copy(outv, out_hbm.at[wid])


def _stage2(valid, karr):
    mesh = plsc.VectorSubcoreMesh(core_axis_name="c", subcore_axis_name="s")
    f = functools.partial(
        pl.kernel,
        out_type=jax.ShapeDtypeStruct((NW, 16), jnp.float32),
        mesh=mesh,
        scratch_types=[
            pltpu.VMEM((L,), jnp.float32),
            pltpu.VMEM((NBINS,), jnp.int32),
            pltpu.VMEM((NBINS,), jnp.float32),
            pltpu.VMEM((64,), jnp.int32),
            pltpu.VMEM((64,), jnp.float32),
            pltpu.VMEM((B,), jnp.int32),
            pltpu.VMEM((16,), jnp.float32),
            pltpu.VMEM((256,), jnp.int32),
            pltpu.VMEM((256,), jnp.float32),
        ],
        compiler_params=pltpu.CompilerParams(needs_layout_passes=False),
    )(_sc_body)
    return f(valid, karr)


def _stage3_body(conc_ref, loss_ref, cmean_ref):
    c = conc_ref[...]
    colmask = lax.broadcasted_iota(jnp.int32, (NW, 16), 1) < ROWS_PER_W
    cm = jnp.where(colmask, c, jnp.float32(0.0))
    cmean_ref[...] = jnp.sum(cm, keepdims=True) / jnp.float32(B)
    loss = jnp.maximum(jnp.float32(TARGET) - c, jnp.float32(0.0))
    loss_ref[...] = jnp.sum(jnp.where(colmask, loss, jnp.float32(0.0)),
                            keepdims=True) / jnp.float32(B)


def _stage3(conc2d):
    return pl.pallas_call(
        _stage3_body,
        out_shape=[
            jax.ShapeDtypeStruct((1, 1), jnp.float32),
            jax.ShapeDtypeStruct((1, 1), jnp.float32),
        ],
    )(conc2d)


def kernel(attention_weights, mask):
    valid, k2d = _stage1(attention_weights, mask)
    conc2d = _stage2(valid, k2d.reshape(B))
    loss2d, cmean2d = _stage3(conc2d)
    return loss2d[0, 0], cmean2d[0, 0]


# speedup vs baseline: 10.2188x; 1.2818x over previous
import functools

import jax
import jax.numpy as jnp
from jax import lax
from jax.experimental import pallas as pl
from jax.experimental.pallas import tpu as pltpu
from jax.experimental.pallas import tpu_sc as plsc

B = 128
L = 32768
TOPK_PCT = 0.2
TARGET = 0.95

NW = 32
ROWS_PER_W = B // NW
NBINS = 4096


def _resolve16(vc, vs, need):
    lane = lax.iota(jnp.int32, 16)
    rc = lax.rev(vc, (0,))
    rs = lax.rev(vs, (0,))
    ccum = jnp.cumsum(rc)
    scum = jnp.cumsum(rs)
    cross = ccum >= need
    jv = plsc.all_reduce_ffs(cross)
    j_s = jv if jv.ndim == 0 else jnp.max(jv)
    sel = lane == j_s
    c_ab = jnp.sum(jnp.where(sel, ccum - rc, 0))
    s_ab = jnp.sum(jnp.where(sel, scum - rs, jnp.float32(0.0)))
    return 15 - j_s, c_ab, s_ab


def _scan_hist4096(cnt_ref, sum_ref, mid_c, mid_s, need):
    lane = lax.iota(jnp.int32, 16)
    lane16 = lane * 16

    @plsc.parallel_loop(0, 16, step=1)
    def _build(j):
        base = j * 256
        vc = jnp.full((16,), 0, jnp.int32)
        vs = jnp.full((16,), 0.0, jnp.float32)
        for w in range(16):
            idx = lane16 + (base + w)
            vc = vc + plsc.load_gather(cnt_ref, [idx])
            vs = vs + plsc.load_gather(sum_ref, [idx])
        mid_c[pl.ds(j * 16, 16)] = vc
        mid_s[pl.ds(j * 16, 16)] = vs

    tc_v = jnp.full((16,), 0, jnp.int32)
    ts_v = jnp.full((16,), 0.0, jnp.float32)
    for w in range(16):
        idx = lane16 + w
        tc_v = tc_v + plsc.load_gather(mid_c, [idx])
        ts_v = ts_v + plsc.load_gather(mid_s, [idx])

    g, c1, s1 = _resolve16(tc_v, ts_v, need)
    r1 = need - c1
    mv_c = mid_c[pl.ds(g * 16, 16)]
    mv_s = mid_s[pl.ds(g * 16, 16)]
    m, c2, s2 = _resolve16(mv_c, mv_s, r1)
    r2 = r1 - c2
    fblk = g * 16 + m
    fv_c = cnt_ref[pl.ds(fblk * 16, 16)]
    fv_s = sum_ref[pl.ds(fblk * 16, 16)]
    p, c3, s3 = _resolve16(fv_c, fv_s, r2)
    return fblk * 16 + p, c1 + c2 + c3, s1 + s2 + s3


def _scan_hist64(cnt_ref, sum_ref, need):
    lane = lax.iota(jnp.int32, 16)
    tc_v = jnp.full((16,), 0, jnp.int32)
    ts_v = jnp.full((16,), 0.0, jnp.float32)
    for i in range(4):
        ci = jnp.sum(cnt_ref[pl.ds(i * 16, 16)])
        si = jnp.sum(sum_ref[pl.ds(i * 16, 16)])
        tc_v = jnp.where(lane == i, ci, tc_v)
        ts_v = jnp.where(lane == i, si, ts_v)
    g, c1, s1 = _resolve16(tc_v, ts_v, need)
    r1 = need - c1
    p, c2, s2 = _resolve16(cnt_ref[pl.ds(g * 16, 16)],
                           sum_ref[pl.ds(g * 16, 16)], r1)
    return g * 16 + p, c1 + c2, s1 + s2


def _sc_body(aw_hbm, mw_hbm, out_hbm, vals, mwords, buf, buf2, cnt_h, sum_h,
             cnt3, sum3, outv, mid_c, mid_s):
    wid = lax.axis_index("s") * 2 + lax.axis_index("c")
    lane = lax.iota(jnp.int32, 16)
    ones_i = jnp.full((16,), 1, jnp.int32)
    zeros_i = jnp.full((16,), 0, jnp.int32)
    zeros_f = jnp.full((16,), 0.0, jnp.float32)
    laned4 = lax.shift_right_logical(lane, 2)
    shvec = jnp.bitwise_and(lane, jnp.int32(3)) * 8

    outv[...] = zeros_f

    for j in range(ROWS_PER_W):
        row = wid * ROWS_PER_W + j
        pltpu.sync_copy(aw_hbm.at[row], vals)
        pltpu.sync_copy(mw_hbm.at[row], mwords)

        @plsc.parallel_loop(0, L // 16, step=1,
                            carry=(jnp.full((16,), 0, jnp.int32),
                                   jnp.full((16,), 0, jnp.int32)))
        def _p0(t, c):
            off_v, acc_v = c
            v = vals[pl.ds(t * 16, 16)]
            w = plsc.load_gather(mwords, [t * 4 + laned4])
            byte = jnp.bitwise_and(lax.shift_right_logical(w, shvec),
                                   jnp.int32(0xFF))
            unmasked = byte == 0
            acc_v = acc_v + jnp.where(unmasked, 1, 0)
            keep = jnp.logical_and(unmasked, v > 0.0)
            ki = jnp.where(keep, 1, 0)
            pos = off_v + jnp.cumsum(ki) - 1
            plsc.store_scatter(buf, [pos], v, mask=keep)
            pc = plsc.all_reduce_population_count(keep)
            return off_v + pc, acc_v

        off_v, acc_v = _p0
        nv = jnp.max(off_v)
        n_valid = jnp.sum(acc_v)
        k = jnp.maximum(
            jnp.int32(1),
            (n_valid.astype(jnp.float32) * jnp.float32(TOPK_PCT))
            .astype(jnp.int32))
        plsc.store_scatter(buf, [jnp.full((16,), nv, jnp.int32) + lane],
                           zeros_f)
        t1 = (nv + 15) // 16

        @plsc.parallel_loop(0, NBINS // 64, step=1)
        def _clr1(t):
            base = t * 64
            for u in range(4):
                cnt_h[pl.ds(base + u * 16, 16)] = zeros_i
                sum_h[pl.ds(base + u * 16, 16)] = zeros_f

        @plsc.parallel_loop(0, t1, step=1)
        def _p1(t):
            v = buf[pl.ds(t * 16, 16)]
            bits = plsc.bitcast(v, jnp.int32)
            b1v = lax.shift_right_logical(bits, 18)
            plsc.addupdate_scatter(cnt_h, [b1v], ones_i)
            plsc.addupdate_scatter(sum_h, [b1v], v)

        b1, c1, s1 = _scan_hist4096(cnt_h, sum_h, mid_c, mid_s, k)
        r = k - c1
        s_above = s1

        @plsc.parallel_loop(0, t1, step=1,
                            carry=jnp.full((16,), 0, jnp.int32))
        def _p2(t, off2):
            v = buf[pl.ds(t * 16, 16)]
            bits = plsc.bitcast(v, jnp.int32)
            m = lax.shift_right_logical(bits, 18) == b1
            mi = jnp.where(m, 1, 0)
            pos = off2 + jnp.cumsum(mi) - 1
            plsc.store_scatter(buf2, [pos], v, mask=m)
            return off2 + plsc.all_reduce_population_count(m)

        m1 = jnp.max(_p2)
        plsc.store_scatter(buf2, [jnp.full((16,), m1, jnp.int32) + lane],
                           zeros_f)
        t2 = (m1 + 15) // 16

        @plsc.parallel_loop(0, NBINS // 64, step=1)
        def _clr2(t):
            base = t * 64
            for u in range(4):
                cnt_h[pl.ds(base + u * 16, 16)] = zeros_i
                sum_h[pl.ds(base + u * 16, 16)] = zeros_f

        @plsc.parallel_loop(0, t2, step=1)
        def _p2h(t):
            v = buf2[pl.ds(t * 16, 16)]
            bits = plsc.bitcast(v, jnp.int32)
            b2v = jnp.bitwise_and(lax.shift_right_logical(bits, 6),
                                  jnp.int32(0xFFF))
            plsc.addupdate_scatter(cnt_h, [b2v], ones_i)
            plsc.addupdate_scatter(sum_h, [b2v], v)

        b2, c2, s2 = _scan_hist4096(cnt_h, sum_h, mid_c, mid_s, r)
        r = r - c2
        s_above = s_above + s2

        pref = jnp.bitwise_or(lax.shift_left(b1, 12), b2)
        for u in range(4):
            cnt3[pl.ds(u * 16, 16)] = zeros_i
            sum3[pl.ds(u * 16, 16)] = zeros_f

        @plsc.parallel_loop(0, t2, step=1)
        def _p3(t):
            v = buf2[pl.ds(t * 16, 16)]
            bits = plsc.bitcast(v, jnp.int32)
            m = lax.shift_right_logical(bits, 6) == pref
            b3v = jnp.bitwise_and(bits, jnp.int32(0x3F))
            plsc.addupdate_scatter(cnt3, [b3v], ones_i, mask=m)
            plsc.addupdate_scatter(sum3, [b3v], v, mask=m)

        b3, c3, s3 = _scan_hist64(cnt3, sum3, r)
        r = r - c3
        s_above = s_above + s3

        vbits = jnp.bitwise_or(
            jnp.bitwise_or(lax.shift_left(b1, 18), lax.shift_left(b2, 6)), b3)
        vf = plsc.bitcast(jnp.full((16,), 1, jnp.int32) * vbits, jnp.float32)
        val = jnp.max(vf)
        conc = s_above + r.astype(jnp.float32) * val
        outv[...] = jnp.where(lane == j, conc, outv[...])

    pltpu.sync_copy(outv, out_hbm.at[wid])


def _stage_sc(aw, mwords):
    mesh = plsc.VectorSubcoreMesh(core_axis_name="c", subcore_axis_name="s")
    f = functools.partial(
        pl.kernel,
        out_type=jax.ShapeDtypeStruct((NW, 16), jnp.float32),
        mesh=mesh,
        scratch_types=[
            pltpu.VMEM((L,), jnp.float32),
            pltpu.VMEM((L // 4,), jnp.int32),
            pltpu.VMEM((L + 16,), jnp.float32),
            pltpu.VMEM((L + 16,), jnp.float32),
            pltpu.VMEM((NBINS,), jnp.int32),
            pltpu.VMEM((NBINS,), jnp.float32),
            pltpu.VMEM((64,), jnp.int32),
            pltpu.VMEM((64,), jnp.float32),
            pltpu.VMEM((16,), jnp.float32),
            pltpu.VMEM((256,), jnp.int32),
            pltpu.VMEM((256,), jnp.float32),
        ],
        compiler_params=pltpu.CompilerParams(needs_layout_passes=False),
    )(_sc_body)
    return f(aw, mwords)


def _stageB_body(conc_ref, loss_ref, cmean_ref):
    c = conc_ref[...]
    colmask = lax.broadcasted_iota(jnp.int32, (NW, 16), 1) < ROWS_PER_W
    cm = jnp.where(colmask, c, jnp.float32(0.0))
    cmean_ref[...] = jnp.sum(cm, keepdims=True) / jnp.float32(B)
    loss = jnp.maximum(jnp.float32(TARGET) - c, jnp.float32(0.0))
    loss_ref[...] = jnp.sum(jnp.where(colmask, loss, jnp.float32(0.0)),
                            keepdims=True) / jnp.float32(B)


def _stageB(conc2d):
    return pl.pallas_call(
        _stageB_body,
        out_shape=[
            jax.ShapeDtypeStruct((1, 1), jnp.float32),
            jax.ShapeDtypeStruct((1, 1), jnp.float32),
        ],
    )(conc2d)


def kernel(attention_weights, mask):
    mwords = lax.bitcast_convert_type(
        mask.astype(jnp.uint8).reshape(B, L // 4, 4), jnp.int32)
    conc2d = _stage_sc(attention_weights, mwords)
    loss2d, cmean2d = _stageB(conc2d)
    return loss2d[0, 0], cmean2d[0, 0]
